# Initial kernel scaffold; baseline (speedup 1.0000x reference)
import functools

import jax
import jax.numpy as jnp
from jax import lax
from jax.experimental import pallas as pl
from jax.experimental.pallas import tpu as pltpu
from jax.experimental.pallas import tpu_sc as plsc

V = 1_000_000
B = 16384
T = 200
L = 16
NS = 16
NC = 2
CPT = B // NS
G = CPT // 128
R = 50


def _body(label_h, text_h, w0_h, w1_h, out0_h, out1_h, lc0_h, lc1_h,
          hist_sh, lcst_sh, labels_v, vals_v, text_v, acc_v, lcred_v):
    c = lax.axis_index("c")
    s = lax.axis_index("s")

    @pl.when(jnp.logical_and(s == 0, c == 0))
    def _():
        pltpu.sync_copy(w0_h, hist_sh)

    @pl.when(jnp.logical_and(s == 0, c == 1))
    def _():
        pltpu.sync_copy(w1_h, hist_sh)

    pltpu.sync_copy(label_h.at[pl.ds(s * CPT, CPT)], labels_v)
    acc = jnp.zeros((L,), jnp.int32)
    for g in range(G):
        for k in range(128 // L):
            lv = labels_v[pl.ds(g * 128 + k * L, L)]
            m = lv == c
            vals_v[g, pl.ds(k * L, L)] = jnp.where(m, 1.0, 0.0).astype(jnp.float32)
            acc = acc + jnp.where(m, 1, 0)
    acc_v[...] = acc

    plsc.subcore_barrier()

    def batch(bi, carry):
        pltpu.sync_copy(text_h.at[pl.ds(bi * R, R), pl.ds(s * G, G), :], text_v)
        for r in range(R):
            pltpu.sync_copy(vals_v, hist_sh.at[text_v.at[r]], add=True)
        return carry

    lax.fori_loop(0, T // R, batch, 0)

    pltpu.sync_copy(acc_v, lcst_sh.at[s])
    plsc.subcore_barrier()

    @pl.when(jnp.logical_and(s == 0, c == 0))
    def _():
        pltpu.sync_copy(hist_sh, out0_h)

    @pl.when(jnp.logical_and(s == 0, c == 1))
    def _():
        pltpu.sync_copy(hist_sh, out1_h)

    @pl.when(s == 1)
    def _():
        pltpu.sync_copy(lcst_sh, lcred_v)
        tot = jnp.zeros((L,), jnp.int32)
        for i in range(NS):
            tot = tot + lcred_v[i, :]
        total = jnp.sum(tot)
        acc_v[...] = jnp.full((L,), total, jnp.int32)

        @pl.when(c == 0)
        def _():
            pltpu.sync_copy(acc_v, lc0_h)

        @pl.when(c == 1)
        def _():
            pltpu.sync_copy(acc_v, lc1_h)


_hist = functools.partial(
    pl.kernel,
    out_type=[
        jax.ShapeDtypeStruct((V,), jnp.float32),
        jax.ShapeDtypeStruct((V,), jnp.float32),
        jax.ShapeDtypeStruct((L,), jnp.int32),
        jax.ShapeDtypeStruct((L,), jnp.int32),
    ],
    mesh=plsc.VectorSubcoreMesh(core_axis_name="c", subcore_axis_name="s"),
    scratch_types=[
        pltpu.VMEM_SHARED((V,), jnp.float32),
        pltpu.VMEM_SHARED((NS, L), jnp.int32),
        pltpu.VMEM((CPT,), jnp.int32),
        pltpu.VMEM((G, 128), jnp.float32),
        pltpu.VMEM((R, G, 128), jnp.int32),
        pltpu.VMEM((L,), jnp.int32),
        pltpu.VMEM((NS, L), jnp.int32),
    ],
)(_body)


def kernel(label, text, w_counts0, w_counts1):
    text3 = text.reshape(T, B // 128, 128)
    w0, w1, lc0v, lc1v = _hist(label.astype(jnp.int32), text3,
                               w_counts0, w_counts1)
    return w0, w1, lc0v[0], lc1v[0]


# speedup vs baseline: 46.6520x; 46.6520x over previous
import functools

import jax
import jax.numpy as jnp
from jax import lax
from jax.experimental import pallas as pl
from jax.experimental.pallas import tpu as pltpu
from jax.experimental.pallas import tpu_sc as plsc

V = 1_000_000
B = 16384
T = 200
L = 16
NS = 16
NC = 2
CPT = B // NS
G = CPT // 128
R = 40


def _body(label_h, text_h, w0_h, w1_h, out0_h, out1_h, lc0_h, lc1_h,
          hist_sh, lcsum_sh, labels_v, vals_v, text_v, accf_v, idx0_v, lcf_v):
    c = lax.axis_index("c")
    s = lax.axis_index("s")

    @pl.when(jnp.logical_and(s == 0, c == 0))
    def _():
        pltpu.sync_copy(w0_h, hist_sh)

    @pl.when(jnp.logical_and(s == 0, c == 1))
    def _():
        pltpu.sync_copy(w1_h, hist_sh)

    pltpu.sync_copy(label_h.at[pl.ds(s * CPT, CPT)], labels_v)
    accf = jnp.zeros((L,), jnp.float32)
    for g in range(G):
        for k in range(128 // L):
            lv = labels_v[pl.ds(g * 128 + k * L, L)]
            mv = jnp.where(lv == c, 1.0, 0.0).astype(jnp.float32)
            vals_v[pl.ds(g * 128 + k * L, L)] = mv
            accf = accf + mv
    accf_v[...] = accf
    idx0_v[...] = jnp.zeros((L,), jnp.int32)

    @pl.when(s == 1)
    def _():
        lcf_v[...] = jnp.zeros((L,), jnp.float32)
        pltpu.sync_copy(lcf_v, lcsum_sh)

    plsc.subcore_barrier()

    pltpu.sync_copy(accf_v, lcsum_sh.at[idx0_v], add=True)

    def batch(bi, carry):
        off = pl.multiple_of(bi * B + s * CPT, CPT)
        pltpu.sync_copy(text_h.at[pl.ds(off, CPT)], text_v)
        pltpu.sync_copy(vals_v, hist_sh.at[text_v], add=True)
        return carry

    lax.fori_loop(0, T, batch, 0)

    plsc.subcore_barrier()

    @pl.when(jnp.logical_and(s == 0, c == 0))
    def _():
        pltpu.sync_copy(hist_sh, out0_h)

    @pl.when(jnp.logical_and(s == 0, c == 1))
    def _():
        pltpu.sync_copy(hist_sh, out1_h)

    @pl.when(jnp.logical_and(s == 1, c == 0))
    def _():
        pltpu.sync_copy(lcsum_sh, lc0_h)

    @pl.when(jnp.logical_and(s == 1, c == 1))
    def _():
        pltpu.sync_copy(lcsum_sh, lc1_h)


_hist = functools.partial(
    pl.kernel,
    out_type=[
        jax.ShapeDtypeStruct((V,), jnp.float32),
        jax.ShapeDtypeStruct((V,), jnp.float32),
        jax.ShapeDtypeStruct((L,), jnp.float32),
        jax.ShapeDtypeStruct((L,), jnp.float32),
    ],
    mesh=plsc.VectorSubcoreMesh(core_axis_name="c", subcore_axis_name="s"),
    scratch_types=[
        pltpu.VMEM_SHARED((V,), jnp.float32),
        pltpu.VMEM_SHARED((L,), jnp.float32),
        pltpu.VMEM((CPT,), jnp.int32),
        pltpu.VMEM((CPT,), jnp.float32),
        pltpu.VMEM((CPT,), jnp.int32),
        pltpu.VMEM((L,), jnp.float32),
        pltpu.VMEM((L,), jnp.int32),
        pltpu.VMEM((L,), jnp.float32),
    ],
)(_body)


def kernel(label, text, w_counts0, w_counts1):
    w0, w1, lc0v, lc1v = _hist(label.astype(jnp.int32),
                               text.astype(jnp.int32).reshape(T * B),
                               w_counts0, w_counts1)
    return w0, w1, lc0v[0].astype(jnp.int32), lc1v[0].astype(jnp.int32)


# speedup vs baseline: 95.2638x; 2.0420x over previous
import functools

import jax
import jax.numpy as jnp
from jax import lax
from jax.experimental import pallas as pl
from jax.experimental.pallas import tpu as pltpu
from jax.experimental.pallas import tpu_sc as plsc

V = 1_000_000
B = 16384
T = 200
L = 16
NS = 16
NC = 2
CPT = B // NS
G = CPT // 128
R = 40


NBUF = 4


def _body(label_h, text_h, w0_h, w1_h, out0_h, out1_h, lc0_h, lc1_h,
          hist_sh, lcsum_sh, labels_v, vals_v, accf_v, idx0_v, lcf_v,
          text_bufs, lsems, ssems):
    c = lax.axis_index("c")
    s = lax.axis_index("s")

    @pl.when(jnp.logical_and(s == 0, c == 0))
    def _():
        pltpu.sync_copy(w0_h, hist_sh)

    @pl.when(jnp.logical_and(s == 0, c == 1))
    def _():
        pltpu.sync_copy(w1_h, hist_sh)

    pltpu.sync_copy(label_h.at[pl.ds(s * CPT, CPT)], labels_v)
    accf = jnp.zeros((L,), jnp.float32)
    for g in range(G):
        for k in range(128 // L):
            lv = labels_v[pl.ds(g * 128 + k * L, L)]
            mv = jnp.where(lv == c, 1.0, 0.0).astype(jnp.float32)
            vals_v[pl.ds(g * 128 + k * L, L)] = mv
            accf = accf + mv
    accf_v[...] = accf
    idx0_v[...] = jnp.zeros((L,), jnp.int32)

    @pl.when(s == 1)
    def _():
        lcf_v[...] = jnp.zeros((L,), jnp.float32)
        pltpu.sync_copy(lcf_v, lcsum_sh)

    plsc.subcore_barrier()

    pltpu.sync_copy(accf_v, lcsum_sh.at[idx0_v], add=True)

    def load(row, j):
        off = pl.multiple_of(row * B + s * CPT, CPT)
        pltpu.async_copy(text_h.at[pl.ds(off, CPT)], text_bufs[j], lsems[j])

    for j in range(NBUF):
        load(j, j)

    nbatch = T // NBUF

    def batch(bi, carry):
        scat = []
        for j in range(NBUF):
            pltpu.make_async_copy(text_h.at[pl.ds(0, CPT)], text_bufs[j],
                                  lsems[j]).wait()
            scat.append(pltpu.async_copy(vals_v, hist_sh.at[text_bufs[j]],
                                         ssems[j], add=True))
        for j in range(NBUF):
            scat[j].wait()

            @pl.when(bi < nbatch - 1)
            def _():
                load((bi + 1) * NBUF + j, j)

        return carry

    lax.fori_loop(0, nbatch, batch, 0)

    plsc.subcore_barrier()

    @pl.when(jnp.logical_and(s == 0, c == 0))
    def _():
        pltpu.sync_copy(hist_sh, out0_h)

    @pl.when(jnp.logical_and(s == 0, c == 1))
    def _():
        pltpu.sync_copy(hist_sh, out1_h)

    @pl.when(jnp.logical_and(s == 1, c == 0))
    def _():
        pltpu.sync_copy(lcsum_sh, lc0_h)

    @pl.when(jnp.logical_and(s == 1, c == 1))
    def _():
        pltpu.sync_copy(lcsum_sh, lc1_h)


_hist = functools.partial(
    pl.kernel,
    out_type=[
        jax.ShapeDtypeStruct((V,), jnp.float32),
        jax.ShapeDtypeStruct((V,), jnp.float32),
        jax.ShapeDtypeStruct((L,), jnp.float32),
        jax.ShapeDtypeStruct((L,), jnp.float32),
    ],
    mesh=plsc.VectorSubcoreMesh(core_axis_name="c", subcore_axis_name="s"),
    scratch_types=[
        pltpu.VMEM_SHARED((V,), jnp.float32),
        pltpu.VMEM_SHARED((L,), jnp.float32),
        pltpu.VMEM((CPT,), jnp.int32),
        pltpu.VMEM((CPT,), jnp.float32),
        pltpu.VMEM((L,), jnp.float32),
        pltpu.VMEM((L,), jnp.int32),
        pltpu.VMEM((L,), jnp.float32),
        [pltpu.VMEM((CPT,), jnp.int32)] * NBUF,
        [pltpu.SemaphoreType.DMA] * NBUF,
        [pltpu.SemaphoreType.DMA] * NBUF,
    ],
)(_body)


def kernel(label, text, w_counts0, w_counts1):
    w0, w1, lc0v, lc1v = _hist(label.astype(jnp.int32),
                               text.astype(jnp.int32).reshape(T * B),
                               w_counts0, w_counts1)
    return w0, w1, lc0v[0].astype(jnp.int32), lc1v[0].astype(jnp.int32)
